# Initial kernel scaffold; baseline (speedup 1.0000x reference)
import jax
import jax.numpy as jnp
from jax.experimental import pallas as pl


def kernel(x, x_ft, edge_index, W_pe, b_pe, W_gos, b_gos, W_prot, b_prot, W_tr, b_tr, W_pr, b_pr, W_eff, b_eff, W_int, b_int, W_conv, b_conv, W_convo, b_convo, W_fls, b_fls, W_drug, b_drug, W_dis, b_dis, W_odrug, b_odrug, W_odis, b_odis):
    raise NotImplementedError("write your pallas kernel here")


# speedup vs baseline: 14.2786x; 14.2786x over previous
import functools

import jax
import jax.numpy as jnp
from jax import lax
from jax.experimental import pallas as pl
from jax.experimental.pallas import tpu as pltpu
from jax.experimental.pallas import tpu_sc as plsc

N = 10000
NPAD = 10240
E = 320000
EPAD = 327680
ECH = EPAD // 128
CPT = ECH // 16
CPW = ECH // 32
H = 128
PE, GOS, PROT = 128, 128, 480

_mesh = plsc.VectorSubcoreMesh(core_axis_name="c", subcore_axis_name="s")


def _act(v):
    return jnp.where(v >= 0, v, 0.1 * v)


@functools.partial(
    pl.kernel,
    out_type=jax.ShapeDtypeStruct((2, NPAD), jnp.float32),
    mesh=_mesh,
    scratch_types=[
        pltpu.VMEM((CPW, 128), jnp.int32),
        pltpu.VMEM((128,), jnp.float32),
        pltpu.VMEM((640,), jnp.float32),
        pltpu.VMEM_SHARED((NPAD,), jnp.float32),
    ],
)
def _deg_kernel(dst_hbm, out_hbm, idx_v, ones_v, zero_v, deg_sh):
    c = lax.axis_index("c")
    s = lax.axis_index("s")
    for k in range(8):
        ones_v[pl.ds(k * 16, 16)] = jnp.ones((16,), jnp.float32)
    for k in range(40):
        zero_v[pl.ds(k * 16, 16)] = jnp.zeros((16,), jnp.float32)
    pltpu.sync_copy(zero_v, deg_sh.at[pl.ds(s * 640, 640)])
    base = (c * 16 + s) * CPW
    pltpu.sync_copy(dst_hbm.at[pl.ds(base, CPW)], idx_v)
    plsc.subcore_barrier()

    def body(j, carry):
        pltpu.sync_copy(ones_v, deg_sh.at[idx_v.at[j]], add=True)
        return carry

    lax.fori_loop(0, CPW, body, 0)
    plsc.subcore_barrier()
    pltpu.sync_copy(deg_sh.at[pl.ds(s * 640, 640)],
                    out_hbm.at[c, pl.ds(s * 640, 640)])


@functools.partial(
    pl.kernel,
    out_type=jax.ShapeDtypeStruct((2, NPAD, H), jnp.float32),
    mesh=_mesh,
    scratch_types=[
        pltpu.VMEM((32, 128), jnp.int32),
        pltpu.VMEM((32, 128), jnp.int32),
        pltpu.VMEM((128, H), jnp.float32),
        pltpu.VMEM_SHARED((NPAD, H), jnp.float32),
        pltpu.SemaphoreType.DMA,
    ],
)
def _edge_kernel(table_hbm, init_hbm, src_hbm, dst_hbm, out_hbm,
                 src_v, dst_v, rows_v, acc_sh, sem):
    c = lax.axis_index("c")
    s = lax.axis_index("s")
    pltpu.sync_copy(init_hbm.at[c, pl.ds(s * 640, 640)],
                    acc_sh.at[pl.ds(s * 640, 640)])
    plsc.subcore_barrier()

    def outer(b, carry):
        base = s * CPT + b * 32
        pltpu.sync_copy(src_hbm.at[c, pl.ds(base, 32)], src_v)
        pltpu.sync_copy(dst_hbm.at[pl.ds(base, 32)], dst_v)

        def body(j, carry2):
            pltpu.async_copy(table_hbm.at[src_v.at[j]], rows_v, sem).wait()
            pltpu.sync_copy(rows_v, acc_sh.at[dst_v.at[j]], add=True)
            return carry2

        lax.fori_loop(0, 32, body, 0)
        return carry

    lax.fori_loop(0, CPT // 32, outer, 0)
    plsc.subcore_barrier()
    pltpu.sync_copy(acc_sh.at[pl.ds(s * 640, 640)],
                    out_hbm.at[c, pl.ds(s * 640, 640)])


def _enc_body(x_ref, xft_ref, deg_ref,
              wpe_ref, bpe_ref, wgos_ref, bgos_ref, wprot_ref, bprot_ref,
              wft_ref, bft_ref, wconv_ref, wconvo_ref,
              table_ref, init_ref, dinv_ref):
    xb = x_ref[...]
    h0 = _act(jnp.dot(xb[:, :PE], wpe_ref[...],
                      preferred_element_type=jnp.float32) + bpe_ref[...])
    h0 += _act(jnp.dot(xb[:, PE:PE + GOS], wgos_ref[...],
                       preferred_element_type=jnp.float32) + bgos_ref[...])
    h0 += _act(jnp.dot(xb[:, PE + GOS:PE + GOS + PROT], wprot_ref[...],
                       preferred_element_type=jnp.float32) + bprot_ref[...])
    pre = jnp.dot(xft_ref[...], wft_ref[...],
                  preferred_element_type=jnp.float32) + bft_ref[...]
    o0 = (_act(pre[:, 0:128]) + _act(pre[:, 128:256])
          + _act(pre[:, 256:384]) + _act(pre[:, 384:512]))
    xw_h = jnp.dot(h0, wconv_ref[...], preferred_element_type=jnp.float32)
    xw_o = jnp.dot(o0, wconvo_ref[...], preferred_element_type=jnp.float32)
    deg = deg_ref[0] + deg_ref[1] + 2.0
    dinv = lax.rsqrt(deg)
    th = xw_h * dinv
    to = xw_o * dinv
    table_ref[0] = th
    table_ref[1] = to
    init_ref[0] = 2.0 * th
    init_ref[1] = 2.0 * to
    dinv_ref[...] = dinv


_BM = 640


def _enc_call(xp, xftp, deg2, W_pe, b_pe, W_gos, b_gos, W_prot, b_prot,
              Wft, bft, W_conv, W_convo):
    grid = NPAD // _BM
    full = lambda i: (0, 0)
    return pl.pallas_call(
        _enc_body,
        grid=(grid,),
        in_specs=[
            pl.BlockSpec((_BM, PE + GOS + PROT), lambda i: (i, 0)),
            pl.BlockSpec((_BM, 64), lambda i: (i, 0)),
            pl.BlockSpec((2, _BM, 1), lambda i: (0, i, 0)),
            pl.BlockSpec((PE, H), full), pl.BlockSpec((1, H), full),
            pl.BlockSpec((GOS, H), full), pl.BlockSpec((1, H), full),
            pl.BlockSpec((PROT, H), full), pl.BlockSpec((1, H), full),
            pl.BlockSpec((64, 512), full), pl.BlockSpec((1, 512), full),
            pl.BlockSpec((H, H), full), pl.BlockSpec((H, H), full),
        ],
        out_specs=[
            pl.BlockSpec((2, _BM, H), lambda i: (0, i, 0)),
            pl.BlockSpec((2, _BM, H), lambda i: (0, i, 0)),
            pl.BlockSpec((_BM, 1), lambda i: (i, 0)),
        ],
        out_shape=[
            jax.ShapeDtypeStruct((2, NPAD, H), jnp.float32),
            jax.ShapeDtypeStruct((2, NPAD, H), jnp.float32),
            jax.ShapeDtypeStruct((NPAD, 1), jnp.float32),
        ],
    )(xp, xftp, deg2, W_pe, b_pe, W_gos, b_gos, W_prot, b_prot,
      Wft, bft, W_conv, W_convo)


def _head_body(acc_ref, dinv_ref, bconv_ref, bconvo_ref,
               wfls_ref, bfls_ref, wdrug_ref, bdrug_ref, wdis_ref, bdis_ref,
               wo4d_ref, wo4g_ref, bo4_ref, out_ref):
    dinv = dinv_ref[...]
    h = _act(acc_ref[0] * dinv + bconv_ref[...])
    ho = _act(acc_ref[1] * dinv + bconvo_ref[...])
    z = _act(jnp.dot(h, wfls_ref[:H], preferred_element_type=jnp.float32)
             + jnp.dot(ho, wfls_ref[H:], preferred_element_type=jnp.float32)
             + bfls_ref[...])
    zd = _act(jnp.dot(z, wdrug_ref[...], preferred_element_type=jnp.float32)
              + bdrug_ref[...])
    zg = _act(jnp.dot(z, wdis_ref[...], preferred_element_type=jnp.float32)
              + bdis_ref[...])
    out_ref[...] = (jnp.dot(zd, wo4d_ref[...], preferred_element_type=jnp.float32)
                    + jnp.dot(zg, wo4g_ref[...], preferred_element_type=jnp.float32)
                    + bo4_ref[...])


_BH = 1000


def _head_call(acc2, dinv, b_conv, b_convo, W_fls, b_fls,
               W_drug, b_drug, W_dis, b_dis, Wo4d, Wo4g, bo4):
    grid = N // _BH
    full = lambda i: (0, 0)
    return pl.pallas_call(
        _head_body,
        grid=(grid,),
        in_specs=[
            pl.BlockSpec((2, _BH, H), lambda i: (0, i, 0)),
            pl.BlockSpec((_BH, 1), lambda i: (i, 0)),
            pl.BlockSpec((1, H), full), pl.BlockSpec((1, H), full),
            pl.BlockSpec((2 * H, H), full), pl.BlockSpec((1, H), full),
            pl.BlockSpec((H, H), full), pl.BlockSpec((1, H), full),
            pl.BlockSpec((H, H), full), pl.BlockSpec((1, H), full),
            pl.BlockSpec((H, 4), full), pl.BlockSpec((H, 4), full),
            pl.BlockSpec((1, 4), full),
        ],
        out_specs=pl.BlockSpec((_BH, 4), lambda i: (i, 0)),
        out_shape=jax.ShapeDtypeStruct((N, 4), jnp.float32),
    )(acc2, dinv, b_conv, b_convo, W_fls, b_fls,
      W_drug, b_drug, W_dis, b_dis, Wo4d, Wo4g, bo4)


def kernel(x, x_ft, edge_index, W_pe, b_pe, W_gos, b_gos, W_prot, b_prot,
           W_tr, b_tr, W_pr, b_pr, W_eff, b_eff, W_int, b_int,
           W_conv, b_conv, W_convo, b_convo, W_fls, b_fls,
           W_drug, b_drug, W_dis, b_dis, W_odrug, b_odrug, W_odis, b_odis):
    src = edge_index[0].astype(jnp.int32)
    dst = edge_index[1].astype(jnp.int32)
    sent = N + (jnp.arange(EPAD - E, dtype=jnp.int32) % (NPAD - N))
    srcp = jnp.concatenate([src, sent]).reshape(ECH, 128)
    dstp = jnp.concatenate([dst, sent]).reshape(ECH, 128)
    src2 = jnp.stack([srcp, srcp + NPAD])

    deg2 = _deg_kernel(dstp)

    xp = jnp.pad(x, ((0, NPAD - N), (0, 0)))
    xftp = jnp.pad(x_ft, ((0, NPAD - N), (0, 64 - 54)))
    Wft = jnp.zeros((64, 512), jnp.float32)
    Wft = Wft.at[0:3, 0:128].set(W_tr).at[3:6, 128:256].set(W_pr)
    Wft = Wft.at[6:30, 256:384].set(W_eff).at[30:54, 384:512].set(W_int)
    bft = jnp.concatenate([b_tr, b_pr, b_eff, b_int]).reshape(1, 512)

    table2, init2, dinv = _enc_call(
        xp, xftp, deg2.reshape(2, NPAD, 1),
        W_pe, b_pe.reshape(1, H), W_gos, b_gos.reshape(1, H),
        W_prot, b_prot.reshape(1, H), Wft, bft, W_conv, W_convo)

    acc2 = _edge_kernel(table2.reshape(2 * NPAD, H), init2, src2, dstp)

    Wo4d = jnp.pad(W_odrug, ((0, 0), (0, 2)))
    Wo4g = jnp.pad(W_odis, ((0, 0), (2, 0)))
    bo4 = jnp.concatenate([b_odrug, b_odis]).reshape(1, 4)
    return _head_call(acc2, dinv, b_conv.reshape(1, H), b_convo.reshape(1, H),
                      W_fls, b_fls.reshape(1, H), W_drug, b_drug.reshape(1, H),
                      W_dis, b_dis.reshape(1, H), Wo4d, Wo4g, bo4)
